# Initial kernel scaffold; baseline (speedup 1.0000x reference)
import jax
import jax.numpy as jnp
from jax.experimental import pallas as pl


def kernel(xtoken_seq, char_seq, target_chars, num_tokens, max_form_len, max_num_tags, eos_id, sep_id, params):
    raise NotImplementedError("write your pallas kernel here")


# speedup vs baseline: 20.2571x; 20.2571x over previous
import functools

import jax
import jax.numpy as jnp
from jax import lax
from jax.experimental import pallas as pl
from jax.experimental.pallas import tpu as pltpu
from jax.experimental.pallas import tpu_sc as plsc

HID = 256
CHUNK = 128


def _prep_body(tc_ref, eos_ref, sep_ref, ct_ref, wc_ref, w0f_ref, w0r_ref,
               b0f_ref, b0r_ref,
               counts_ref, ids_ref, cs_ref, ptf_ref, ptr_ref):
    tc = tc_ref[:]
    n, l = tc.shape
    eos_id = eos_ref[0]
    sep_id = sep_ref[0]
    tri = (lax.broadcasted_iota(jnp.int32, (l, l), 0)
           <= lax.broadcasted_iota(jnp.int32, (l, l), 1)).astype(jnp.float32)
    pos = lax.broadcasted_iota(jnp.int32, (n, l), 1)
    eos = (tc == eos_id) | (pos == l - 1)
    cs_eos = jnp.dot(eos.astype(jnp.float32), tri,
                     preferred_element_type=jnp.float32)
    first_eos = (cs_eos == 1.0) & eos
    cs_fe = jnp.dot(first_eos.astype(jnp.float32), tri,
                    preferred_element_type=jnp.float32)
    sep = (tc == sep_id) & (cs_fe == 0.0)
    tag = first_eos | sep
    tagf = tag.astype(jnp.float32)
    out_cols = jnp.dot(tagf, tri, preferred_element_type=jnp.float32) - 1.0
    counts_ref[:] = jnp.sum(tagf, axis=1, keepdims=True).astype(jnp.int32)
    ocm = jnp.where(tag, out_cols, -1.0)
    tcf = tc.astype(jnp.float32)
    for c in range(l):
        col = jnp.sum(jnp.where(ocm == float(c), tcf, 0.0),
                      axis=1, keepdims=True)
        ids_ref[:, c:c + 1] = col.astype(jnp.int32)
    cs_ref[:] = jnp.dot(ct_ref[:], wc_ref[:],
                        preferred_element_type=jnp.float32)
    ptf_ref[:] = jnp.dot(ct_ref[:], w0f_ref[:],
                         preferred_element_type=jnp.float32) + b0f_ref[:]
    ptr_ref[:] = jnp.dot(ct_ref[:], w0r_ref[:],
                         preferred_element_type=jnp.float32) + b0r_ref[:]


def _run_prep(tc, eos_id, sep_id, ct, wc, w0fT, w0rT, b0f, b0r):
    n, l = tc.shape
    cv, emb = ct.shape
    h4 = w0fT.shape[1]
    smem = pl.BlockSpec(memory_space=pltpu.SMEM)
    return pl.pallas_call(
        _prep_body,
        in_specs=[pl.BlockSpec((n, l), lambda: (0, 0)), smem, smem] +
                 [pl.BlockSpec(x.shape, lambda: (0, 0))
                  for x in (ct, wc, w0fT, w0rT, b0f, b0r)],
        out_specs=[pl.BlockSpec((n, 1), lambda: (0, 0)),
                   pl.BlockSpec((n, l), lambda: (0, 0)),
                   pl.BlockSpec((cv, cv), lambda: (0, 0)),
                   pl.BlockSpec((cv, h4), lambda: (0, 0)),
                   pl.BlockSpec((cv, h4), lambda: (0, 0))],
        out_shape=[jax.ShapeDtypeStruct((n, 1), jnp.int32),
                   jax.ShapeDtypeStruct((n, l), jnp.int32),
                   jax.ShapeDtypeStruct((cv, cv), jnp.float32),
                   jax.ShapeDtypeStruct((cv, h4), jnp.float32),
                   jax.ShapeDtypeStruct((cv, h4), jnp.float32)],
    )(tc, eos_id, sep_id, ct, wc, w0fT, w0rT, b0f, b0r)


def _morph_scores_sc(cs_tab, idx_flat):
    b = idx_flat.shape[0]
    cv = cs_tab.shape[1]
    info = plsc.get_sparse_core_info()
    nw = info.num_cores * info.num_subcores
    b_per_w = b // nw
    ch = min(512, b_per_w)
    mesh = plsc.VectorSubcoreMesh(core_axis_name="c", subcore_axis_name="s")

    @functools.partial(
        pl.kernel, mesh=mesh,
        out_type=jax.ShapeDtypeStruct((b, cv), jnp.float32),
        scratch_types=[pltpu.VMEM((ch,), jnp.int32),
                       pltpu.VMEM((ch, cv), jnp.float32),
                       pltpu.SemaphoreType.DMA],
    )
    def k(cs_hbm, idx_hbm, out_hbm, idx_v, rows_v, sem):
        wid = lax.axis_index("s") * info.num_cores + lax.axis_index("c")
        base = wid * b_per_w

        def chunk(j, carry):
            off = base + j * ch
            pltpu.sync_copy(idx_hbm.at[pl.ds(off, ch)], idx_v)
            pltpu.async_copy(cs_hbm.at[idx_v], rows_v, sem).wait()
            pltpu.sync_copy(rows_v, out_hbm.at[pl.ds(off, ch)])
            return carry

        lax.fori_loop(0, b_per_w // ch, chunk, 0)

    return k(cs_tab, idx_flat)


def _gates(gsum, c_prev):
    h = HID
    i_ = jax.nn.sigmoid(gsum[:, 0:h])
    f_ = jax.nn.sigmoid(gsum[:, h:2 * h])
    g_ = jnp.tanh(gsum[:, 2 * h:3 * h])
    o_ = jax.nn.sigmoid(gsum[:, 3 * h:4 * h])
    c_new = f_ * c_prev + i_ * g_
    h_new = o_ * jnp.tanh(c_new)
    return h_new, c_new


def _l0_body(idsf_ref, idsr_ref, cntf_ref, cntr_ref, ptf_ref, ptr_ref,
             whf_ref, whr_ref, h0f_ref, h0b_ref, st_ref):
    g = pl.program_id(0)

    @pl.when(g == 0)
    def _():
        st_ref[:] = jnp.zeros_like(st_ref)

    def row_body(k, carry):
        hf, cf, hr, cr = carry
        nf = cntf_ref[k, 0]
        kr = CHUNK - 1 - k
        nr = cntr_ref[kr, 0]
        cm = jnp.maximum(nf, nr)

        def step(c, carry2):
            hf, cf, hr, cr = carry2
            idf = idsf_ref[k, c]
            gf = ptf_ref[pl.ds(idf, 1), :] + jnp.dot(
                hf, whf_ref[:], preferred_element_type=jnp.float32)
            hf_new, cf_new = _gates(gf, cf)
            okf = c < nf

            @pl.when(okf)
            def _():
                h0f_ref[pl.ds(k, 1), pl.ds(c, 1), :] = hf_new.reshape(
                    1, 1, HID)

            hf = jnp.where(okf, hf_new, hf)
            cf = jnp.where(okf, cf_new, cf)
            cc = nr - 1 - c
            okr = c < nr
            idr = idsr_ref[kr, jnp.maximum(cc, 0)]
            gr = ptr_ref[pl.ds(idr, 1), :] + jnp.dot(
                hr, whr_ref[:], preferred_element_type=jnp.float32)
            hr_new, cr_new = _gates(gr, cr)

            @pl.when(okr)
            def _():
                h0b_ref[pl.ds(kr, 1), pl.ds(jnp.maximum(cc, 0), 1), :] = (
                    hr_new.reshape(1, 1, HID))

            hr = jnp.where(okr, hr_new, hr)
            cr = jnp.where(okr, cr_new, cr)
            return hf, cf, hr, cr

        return lax.fori_loop(0, cm, step, (hf, cf, hr, cr))

    carry = (st_ref[0:1, :], st_ref[1:2, :], st_ref[2:3, :], st_ref[3:4, :])
    hf, cf, hr, cr = lax.fori_loop(0, CHUNK, row_body, carry)
    st_ref[0:1, :] = hf
    st_ref[1:2, :] = cf
    st_ref[2:3, :] = hr
    st_ref[3:4, :] = cr


def _run_l0(ids, counts, ptf, ptr, whfT, whrT):
    n, l = ids.shape
    grid = n // CHUNK
    h4 = 4 * HID
    smem = pltpu.SMEM
    out_sh = jax.ShapeDtypeStruct((n, l, HID), jnp.float32)
    return pl.pallas_call(
        _l0_body,
        grid=(grid,),
        in_specs=[
            pl.BlockSpec((CHUNK, l), lambda g: (g, 0), memory_space=smem),
            pl.BlockSpec((CHUNK, l), lambda g: (grid - 1 - g, 0),
                         memory_space=smem),
            pl.BlockSpec((CHUNK, 1), lambda g: (g, 0), memory_space=smem),
            pl.BlockSpec((CHUNK, 1), lambda g: (grid - 1 - g, 0),
                         memory_space=smem),
            pl.BlockSpec((128, h4), lambda g: (0, 0)),
            pl.BlockSpec((128, h4), lambda g: (0, 0)),
            pl.BlockSpec((HID, h4), lambda g: (0, 0)),
            pl.BlockSpec((HID, h4), lambda g: (0, 0)),
        ],
        out_specs=[
            pl.BlockSpec((CHUNK, l, HID), lambda g: (g, 0, 0)),
            pl.BlockSpec((CHUNK, l, HID), lambda g: (grid - 1 - g, 0, 0)),
        ],
        out_shape=[out_sh, out_sh],
        scratch_shapes=[pltpu.VMEM((8, HID), jnp.float32)],
        compiler_params=pltpu.CompilerParams(
            dimension_semantics=("arbitrary",)),
    )(ids, ids, counts, counts, ptf, ptr, whfT, whrT)


def _l1_body(h0ff_ref, h0bf_ref, h0fr_ref, h0br_ref, cntf_ref, cntr_ref,
             wif_ref, wir_ref, b1f_ref, b1r_ref, whf_ref, whr_ref,
             h1f_ref, h1b_ref, st_ref):
    g = pl.program_id(0)

    @pl.when(g == 0)
    def _():
        st_ref[:] = jnp.zeros_like(st_ref)

    def xin(href_a, href_b, k, c):
        a = href_a[pl.ds(k, 1), pl.ds(c, 1), :].reshape(1, HID)
        b = href_b[pl.ds(k, 1), pl.ds(c, 1), :].reshape(1, HID)
        return jnp.concatenate([a, b], axis=1)

    def row_body(k, carry):
        hf, cf, hr, cr = carry
        nf = cntf_ref[k, 0]
        kr = CHUNK - 1 - k
        nr = cntr_ref[kr, 0]
        cm = jnp.maximum(nf, nr)

        def step(c, carry2):
            hf, cf, hr, cr = carry2
            xf = xin(h0ff_ref, h0bf_ref, k, c)
            gf = (jnp.dot(xf, wif_ref[:], preferred_element_type=jnp.float32)
                  + b1f_ref[:]
                  + jnp.dot(hf, whf_ref[:],
                            preferred_element_type=jnp.float32))
            hf_new, cf_new = _gates(gf, cf)
            okf = c < nf

            @pl.when(okf)
            def _():
                h1f_ref[pl.ds(k, 1), pl.ds(c, 1), :] = hf_new.reshape(
                    1, 1, HID)

            hf = jnp.where(okf, hf_new, hf)
            cf = jnp.where(okf, cf_new, cf)

            cc = jnp.maximum(nr - 1 - c, 0)
            okr = c < nr
            xr = xin(h0fr_ref, h0br_ref, kr, cc)
            gr = (jnp.dot(xr, wir_ref[:], preferred_element_type=jnp.float32)
                  + b1r_ref[:]
                  + jnp.dot(hr, whr_ref[:],
                            preferred_element_type=jnp.float32))
            hr_new, cr_new = _gates(gr, cr)

            @pl.when(okr)
            def _():
                h1b_ref[pl.ds(kr, 1), pl.ds(cc, 1), :] = hr_new.reshape(
                    1, 1, HID)

            hr = jnp.where(okr, hr_new, hr)
            cr = jnp.where(okr, cr_new, cr)
            return hf, cf, hr, cr

        return lax.fori_loop(0, cm, step, (hf, cf, hr, cr))

    carry = (st_ref[0:1, :], st_ref[1:2, :], st_ref[2:3, :], st_ref[3:4, :])
    hf, cf, hr, cr = lax.fori_loop(0, CHUNK, row_body, carry)
    st_ref[0:1, :] = hf
    st_ref[1:2, :] = cf
    st_ref[2:3, :] = hr
    st_ref[3:4, :] = cr


def _run_l1(h0f, h0b, counts, wi1fT, wi1rT, b1f, b1r, wh1fT, wh1rT):
    n, l, _ = h0f.shape
    grid = n // CHUNK
    h4 = 4 * HID
    smem = pltpu.SMEM
    blk = pl.BlockSpec((CHUNK, l, HID), lambda g: (g, 0, 0))
    blk_r = pl.BlockSpec((CHUNK, l, HID), lambda g: (grid - 1 - g, 0, 0))
    out_sh = jax.ShapeDtypeStruct((n, l, HID), jnp.float32)
    return pl.pallas_call(
        _l1_body,
        grid=(grid,),
        in_specs=[
            blk, blk, blk_r, blk_r,
            pl.BlockSpec((CHUNK, 1), lambda g: (g, 0), memory_space=smem),
            pl.BlockSpec((CHUNK, 1), lambda g: (grid - 1 - g, 0),
                         memory_space=smem),
            pl.BlockSpec((2 * HID, h4), lambda g: (0, 0)),
            pl.BlockSpec((2 * HID, h4), lambda g: (0, 0)),
            pl.BlockSpec((1, h4), lambda g: (0, 0)),
            pl.BlockSpec((1, h4), lambda g: (0, 0)),
            pl.BlockSpec((HID, h4), lambda g: (0, 0)),
            pl.BlockSpec((HID, h4), lambda g: (0, 0)),
        ],
        out_specs=[
            pl.BlockSpec((CHUNK, l, HID), lambda g: (g, 0, 0)),
            pl.BlockSpec((CHUNK, l, HID), lambda g: (grid - 1 - g, 0, 0)),
        ],
        out_shape=[out_sh, out_sh],
        scratch_shapes=[pltpu.VMEM((8, HID), jnp.float32)],
        compiler_params=pltpu.CompilerParams(
            dimension_semantics=("arbitrary",)),
    )(h0f, h0b, h0f, h0b, counts, counts,
      wi1fT, wi1rT, b1f, b1r, wh1fT, wh1rT)


def _proj_body(h1f_ref, h1b_ref, cnt_ref, wo_ref, bo_ref, out_ref):
    n, l, h = h1f_ref.shape
    cnt = cnt_ref[:]
    for c in range(l):
        a = jnp.concatenate([h1f_ref[:, c, :].reshape(n, h),
                             h1b_ref[:, c, :].reshape(n, h)], axis=1)
        y = jnp.dot(a, wo_ref[:],
                    preferred_element_type=jnp.float32) + bo_ref[:]
        out_ref[:, c, :] = jnp.where(cnt > c, y, 0.0)


def _run_proj(h1f, h1b, counts, woT, bo):
    n, l, h = h1f.shape
    grid = n // CHUNK
    out_dim = woT.shape[1]
    blk = pl.BlockSpec((CHUNK, l, HID), lambda g: (g, 0, 0))
    return pl.pallas_call(
        _proj_body,
        grid=(grid,),
        in_specs=[
            blk, blk,
            pl.BlockSpec((CHUNK, 1), lambda g: (g, 0)),
            pl.BlockSpec((2 * HID, out_dim), lambda g: (0, 0)),
            pl.BlockSpec((1, out_dim), lambda g: (0, 0)),
        ],
        out_specs=pl.BlockSpec((CHUNK, l, out_dim), lambda g: (g, 0, 0)),
        out_shape=jax.ShapeDtypeStruct((n, l, out_dim), jnp.float32),
    )(h1f, h1b, counts, woT, bo)


def kernel(xtoken_seq, char_seq, target_chars, num_tokens, max_form_len,
           max_num_tags, eos_id, sep_id, params):
    p = params
    tc = target_chars.astype(jnp.int32)
    n, l = tc.shape
    eos_a = jnp.asarray(eos_id, jnp.int32).reshape(1)
    sep_a = jnp.asarray(sep_id, jnp.int32).reshape(1)

    ct = p['char_table']
    w0fT = p['l0_f_Wih'].T
    w0rT = p['l0_r_Wih'].T
    b0f = (p['l0_f_bih'] + p['l0_f_bhh']).reshape(1, -1)
    b0r = (p['l0_r_bih'] + p['l0_r_bhh']).reshape(1, -1)

    counts, ids, cs_tab, ptf, ptr = _run_prep(
        tc, eos_a, sep_a, ct, p['W_char'], w0fT, w0rT, b0f, b0r)

    scores_flat = _morph_scores_sc(cs_tab, tc.reshape(-1))
    morph_scores = scores_flat.reshape(n, l, -1)

    h0f, h0b = _run_l0(ids, counts, ptf, ptr,
                       p['l0_f_Whh'].T, p['l0_r_Whh'].T)

    b1f = (p['l1_f_bih'] + p['l1_f_bhh']).reshape(1, -1)
    b1r = (p['l1_r_bih'] + p['l1_r_bhh']).reshape(1, -1)
    h1f, h1b = _run_l1(h0f, h0b, counts,
                       p['l1_f_Wih'].T, p['l1_r_Wih'].T, b1f, b1r,
                       p['l1_f_Whh'].T, p['l1_r_Whh'].T)

    padded = _run_proj(h1f, h1b, counts, p['W_out'].T,
                       p['b_out'].reshape(1, -1))
    return morph_scores, padded


# speedup vs baseline: 35.6601x; 1.7604x over previous
import functools

import jax
import jax.numpy as jnp
from jax import lax
from jax.experimental import pallas as pl
from jax.experimental.pallas import tpu as pltpu
from jax.experimental.pallas import tpu_sc as plsc

HID = 256
MAXT = 16
CHUNK = 64


def _prep_body(tc_ref, eos_ref, sep_ref, ct_ref, wc_ref, w0f_ref, w0r_ref,
               b0f_ref, b0r_ref,
               counts_ref, ids_ref, cs_ref, ptf_ref, ptr_ref):
    tc = tc_ref[:]
    n, l = tc.shape
    eos_id = eos_ref[0]
    sep_id = sep_ref[0]
    tri = (lax.broadcasted_iota(jnp.int32, (l, l), 0)
           <= lax.broadcasted_iota(jnp.int32, (l, l), 1)).astype(jnp.float32)
    pos = lax.broadcasted_iota(jnp.int32, (n, l), 1)
    eos = (tc == eos_id) | (pos == l - 1)
    cs_eos = jnp.dot(eos.astype(jnp.float32), tri,
                     preferred_element_type=jnp.float32)
    first_eos = (cs_eos == 1.0) & eos
    cs_fe = jnp.dot(first_eos.astype(jnp.float32), tri,
                    preferred_element_type=jnp.float32)
    sep = (tc == sep_id) & (cs_fe == 0.0)
    tag = first_eos | sep
    tagf = tag.astype(jnp.float32)
    out_cols = jnp.dot(tagf, tri, preferred_element_type=jnp.float32) - 1.0
    counts_ref[:] = jnp.sum(tagf, axis=1, keepdims=True).astype(jnp.int32)
    ocm = jnp.where(tag, out_cols, -1.0)
    tcf = tc.astype(jnp.float32)
    for c in range(l):
        col = jnp.sum(jnp.where(ocm == float(c), tcf, 0.0),
                      axis=1, keepdims=True)
        ids_ref[:, c:c + 1] = col.astype(jnp.int32)
    cs_ref[:] = jnp.dot(ct_ref[:], wc_ref[:],
                        preferred_element_type=jnp.float32)
    ptf_ref[:] = jnp.dot(ct_ref[:], w0f_ref[:],
                         preferred_element_type=jnp.float32) + b0f_ref[:]
    ptr_ref[:] = jnp.dot(ct_ref[:], w0r_ref[:],
                         preferred_element_type=jnp.float32) + b0r_ref[:]


def _run_prep(tc, eos_id, sep_id, ct, wc, w0fT, w0rT, b0f, b0r):
    n, l = tc.shape
    cv, emb = ct.shape
    h4 = w0fT.shape[1]
    smem = pl.BlockSpec(memory_space=pltpu.SMEM)
    return pl.pallas_call(
        _prep_body,
        in_specs=[pl.BlockSpec((n, l), lambda: (0, 0)), smem, smem] +
                 [pl.BlockSpec(x.shape, lambda: (0, 0))
                  for x in (ct, wc, w0fT, w0rT, b0f, b0r)],
        out_specs=[pl.BlockSpec((n, 1), lambda: (0, 0)),
                   pl.BlockSpec((n, l), lambda: (0, 0)),
                   pl.BlockSpec((cv, cv), lambda: (0, 0)),
                   pl.BlockSpec((cv, h4), lambda: (0, 0)),
                   pl.BlockSpec((cv, h4), lambda: (0, 0))],
        out_shape=[jax.ShapeDtypeStruct((n, 1), jnp.int32),
                   jax.ShapeDtypeStruct((n, l), jnp.int32),
                   jax.ShapeDtypeStruct((cv, cv), jnp.float32),
                   jax.ShapeDtypeStruct((cv, h4), jnp.float32),
                   jax.ShapeDtypeStruct((cv, h4), jnp.float32)],
    )(tc, eos_id, sep_id, ct, wc, w0fT, w0rT, b0f, b0r)


def _morph_scores_sc(cs_tab, idx_flat):
    b = idx_flat.shape[0]
    cv = cs_tab.shape[1]
    info = plsc.get_sparse_core_info()
    nw = info.num_cores * info.num_subcores
    b_per_w = b // nw
    ch = min(512, b_per_w)
    mesh = plsc.VectorSubcoreMesh(core_axis_name="c", subcore_axis_name="s")

    @functools.partial(
        pl.kernel, mesh=mesh,
        out_type=jax.ShapeDtypeStruct((b, cv), jnp.float32),
        scratch_types=[pltpu.VMEM((ch,), jnp.int32),
                       pltpu.VMEM((ch, cv), jnp.float32),
                       pltpu.SemaphoreType.DMA],
    )
    def k(cs_hbm, idx_hbm, out_hbm, idx_v, rows_v, sem):
        wid = lax.axis_index("s") * info.num_cores + lax.axis_index("c")
        base = wid * b_per_w

        def chunk(j, carry):
            off = base + j * ch
            pltpu.sync_copy(idx_hbm.at[pl.ds(off, ch)], idx_v)
            pltpu.async_copy(cs_hbm.at[idx_v], rows_v, sem).wait()
            pltpu.sync_copy(rows_v, out_hbm.at[pl.ds(off, ch)])
            return carry

        lax.fori_loop(0, b_per_w // ch, chunk, 0)

    return k(cs_tab, idx_flat)


def _gates(gsum, c_prev):
    h = HID
    i_ = jax.nn.sigmoid(gsum[:, 0:h])
    f_ = jax.nn.sigmoid(gsum[:, h:2 * h])
    g_ = jnp.tanh(gsum[:, 2 * h:3 * h])
    o_ = jax.nn.sigmoid(gsum[:, 3 * h:4 * h])
    c_new = f_ * c_prev + i_ * g_
    h_new = o_ * jnp.tanh(c_new)
    return h_new, c_new


def _gates2(gsum2, c2_prev):
    h = HID
    i_ = jax.nn.sigmoid(gsum2[:, 0:h])
    f_ = jax.nn.sigmoid(gsum2[:, h:2 * h])
    g_ = jnp.tanh(gsum2[:, 2 * h:3 * h])
    o_ = jax.nn.sigmoid(gsum2[:, 3 * h:4 * h])
    c_new = f_ * c2_prev + i_ * g_
    h_new = o_ * jnp.tanh(c_new)
    return h_new, c_new


def _sel2(ok_f, ok_r):
    si = lax.broadcasted_iota(jnp.int32, (2, 1), 0)
    return ((si == 0) & ok_f) | ((si == 1) & ok_r)


def _l0_body(idsf_ref, idsr_ref, cntf_ref, cntr_ref, ptf_ref, ptr_ref,
             wh_ref, h0f_ref, h0b_ref, st_ref):
    g = pl.program_id(0)
    h4 = 4 * HID

    @pl.when(g == 0)
    def _():
        st_ref[:] = jnp.zeros_like(st_ref)

    def row_body(k, carry):
        h2, c2 = carry
        nf = cntf_ref[k, 0]
        kr = CHUNK - 1 - k
        nr = cntr_ref[kr, 0]
        cm = jnp.maximum(nf, nr)

        def step(c, carry2):
            h2, c2 = carry2
            idf = idsf_ref[k, c]
            cc = jnp.maximum(nr - 1 - c, 0)
            idr = idsr_ref[kr, cc]
            pt2 = jnp.concatenate([ptf_ref[pl.ds(idf, 1), :],
                                   ptr_ref[pl.ds(idr, 1), :]], axis=0)
            gq = jnp.dot(h2, wh_ref[:], preferred_element_type=jnp.float32)
            gsum2 = jnp.concatenate([gq[0:1, 0:h4], gq[1:2, h4:2 * h4]],
                                    axis=0) + pt2
            h2n, c2n = _gates2(gsum2, c2)
            okf = c < nf
            okr = c < nr

            @pl.when(okf)
            def _():
                h0f_ref[pl.ds(k, 1), pl.ds(c, 1), :] = h2n[0:1, :].reshape(
                    1, 1, HID)

            @pl.when(okr)
            def _():
                h0b_ref[pl.ds(kr, 1), pl.ds(cc, 1), :] = h2n[1:2, :].reshape(
                    1, 1, HID)

            m2 = _sel2(okf, okr)
            return jnp.where(m2, h2n, h2), jnp.where(m2, c2n, c2)

        return lax.fori_loop(0, cm, step, (h2, c2))

    h2, c2 = lax.fori_loop(0, CHUNK, row_body,
                           (st_ref[0:2, :], st_ref[2:4, :]))
    st_ref[0:2, :] = h2
    st_ref[2:4, :] = c2


def _run_l0(ids, counts, ptf, ptr, whcat):
    n, l = ids.shape
    grid = n // CHUNK
    h4 = 4 * HID
    smem = pltpu.SMEM
    out_sh = jax.ShapeDtypeStruct((n, l, HID), jnp.float32)
    return pl.pallas_call(
        _l0_body,
        grid=(grid,),
        in_specs=[
            pl.BlockSpec((CHUNK, l), lambda g: (g, 0), memory_space=smem),
            pl.BlockSpec((CHUNK, l), lambda g: (grid - 1 - g, 0),
                         memory_space=smem),
            pl.BlockSpec((CHUNK, 1), lambda g: (g, 0), memory_space=smem),
            pl.BlockSpec((CHUNK, 1), lambda g: (grid - 1 - g, 0),
                         memory_space=smem),
            pl.BlockSpec((128, h4), lambda g: (0, 0)),
            pl.BlockSpec((128, h4), lambda g: (0, 0)),
            pl.BlockSpec((HID, 2 * h4), lambda g: (0, 0)),
        ],
        out_specs=[
            pl.BlockSpec((CHUNK, l, HID), lambda g: (g, 0, 0)),
            pl.BlockSpec((CHUNK, l, HID), lambda g: (grid - 1 - g, 0, 0)),
        ],
        out_shape=[out_sh, out_sh],
        scratch_shapes=[pltpu.VMEM((4, HID), jnp.float32)],
        compiler_params=pltpu.CompilerParams(
            dimension_semantics=("arbitrary",)),
    )(ids, ids, counts, counts, ptf, ptr, whcat)


def _l1_body(h0ff_ref, h0bf_ref, h0fr_ref, h0br_ref, cntf_ref, cntr_ref,
             wif_ref, wir_ref, b1f_ref, b1r_ref, wh_ref,
             h1f_ref, h1b_ref, st_ref, xpf_ref, xpr_ref):
    g = pl.program_id(0)
    h4 = 4 * HID
    nl = CHUNK * MAXT

    @pl.when(g == 0)
    def _():
        st_ref[:] = jnp.zeros_like(st_ref)

    af = jnp.concatenate([h0ff_ref[:].reshape(nl, HID),
                          h0bf_ref[:].reshape(nl, HID)], axis=1)
    xpf_ref[:] = jnp.dot(af, wif_ref[:],
                         preferred_element_type=jnp.float32) + b1f_ref[:]
    ar = jnp.concatenate([h0fr_ref[:].reshape(nl, HID),
                          h0br_ref[:].reshape(nl, HID)], axis=1)
    xpr_ref[:] = jnp.dot(ar, wir_ref[:],
                         preferred_element_type=jnp.float32) + b1r_ref[:]

    def row_body(k, carry):
        h2, c2 = carry
        nf = cntf_ref[k, 0]
        kr = CHUNK - 1 - k
        nr = cntr_ref[kr, 0]
        cm = jnp.maximum(nf, nr)

        def step(c, carry2):
            h2, c2 = carry2
            cc = jnp.maximum(nr - 1 - c, 0)
            xp2 = jnp.concatenate(
                [xpf_ref[pl.ds(k * MAXT + c, 1), :],
                 xpr_ref[pl.ds(kr * MAXT + cc, 1), :]], axis=0)
            gq = jnp.dot(h2, wh_ref[:], preferred_element_type=jnp.float32)
            gsum2 = jnp.concatenate([gq[0:1, 0:h4], gq[1:2, h4:2 * h4]],
                                    axis=0) + xp2
            h2n, c2n = _gates2(gsum2, c2)
            okf = c < nf
            okr = c < nr

            @pl.when(okf)
            def _():
                h1f_ref[pl.ds(k, 1), pl.ds(c, 1), :] = h2n[0:1, :].reshape(
                    1, 1, HID)

            @pl.when(okr)
            def _():
                h1b_ref[pl.ds(kr, 1), pl.ds(cc, 1), :] = h2n[1:2, :].reshape(
                    1, 1, HID)

            m2 = _sel2(okf, okr)
            return jnp.where(m2, h2n, h2), jnp.where(m2, c2n, c2)

        return lax.fori_loop(0, cm, step, (h2, c2))

    h2, c2 = lax.fori_loop(0, CHUNK, row_body,
                           (st_ref[0:2, :], st_ref[2:4, :]))
    st_ref[0:2, :] = h2
    st_ref[2:4, :] = c2


def _run_l1(h0f, h0b, counts, wi1fT, wi1rT, b1f, b1r, whcat):
    n, l, _ = h0f.shape
    grid = n // CHUNK
    h4 = 4 * HID
    smem = pltpu.SMEM
    blk = pl.BlockSpec((CHUNK, l, HID), lambda g: (g, 0, 0))
    blk_r = pl.BlockSpec((CHUNK, l, HID), lambda g: (grid - 1 - g, 0, 0))
    out_sh = jax.ShapeDtypeStruct((n, l, HID), jnp.float32)
    return pl.pallas_call(
        _l1_body,
        grid=(grid,),
        in_specs=[
            blk, blk, blk_r, blk_r,
            pl.BlockSpec((CHUNK, 1), lambda g: (g, 0), memory_space=smem),
            pl.BlockSpec((CHUNK, 1), lambda g: (grid - 1 - g, 0),
                         memory_space=smem),
            pl.BlockSpec((2 * HID, h4), lambda g: (0, 0)),
            pl.BlockSpec((2 * HID, h4), lambda g: (0, 0)),
            pl.BlockSpec((1, h4), lambda g: (0, 0)),
            pl.BlockSpec((1, h4), lambda g: (0, 0)),
            pl.BlockSpec((HID, 2 * h4), lambda g: (0, 0)),
        ],
        out_specs=[
            pl.BlockSpec((CHUNK, l, HID), lambda g: (g, 0, 0)),
            pl.BlockSpec((CHUNK, l, HID), lambda g: (grid - 1 - g, 0, 0)),
        ],
        out_shape=[out_sh, out_sh],
        scratch_shapes=[pltpu.VMEM((4, HID), jnp.float32),
                        pltpu.VMEM((CHUNK * MAXT, h4), jnp.float32),
                        pltpu.VMEM((CHUNK * MAXT, h4), jnp.float32)],
        compiler_params=pltpu.CompilerParams(
            dimension_semantics=("arbitrary",)),
    )(h0f, h0b, h0f, h0b, counts, counts, wi1fT, wi1rT, b1f, b1r, whcat)


def _proj_body(h1f_ref, h1b_ref, cnt_ref, wo_ref, bo_ref, out_ref):
    n, l, h = h1f_ref.shape
    cnt = cnt_ref[:]
    for c in range(l):
        a = jnp.concatenate([h1f_ref[:, c, :].reshape(n, h),
                             h1b_ref[:, c, :].reshape(n, h)], axis=1)
        y = jnp.dot(a, wo_ref[:],
                    preferred_element_type=jnp.float32) + bo_ref[:]
        out_ref[:, c, :] = jnp.where(cnt > c, y, 0.0)


def _run_proj(h1f, h1b, counts, woT, bo):
    n, l, h = h1f.shape
    grid = n // CHUNK
    out_dim = woT.shape[1]
    blk = pl.BlockSpec((CHUNK, l, HID), lambda g: (g, 0, 0))
    return pl.pallas_call(
        _proj_body,
        grid=(grid,),
        in_specs=[
            blk, blk,
            pl.BlockSpec((CHUNK, 1), lambda g: (g, 0)),
            pl.BlockSpec((2 * HID, out_dim), lambda g: (0, 0)),
            pl.BlockSpec((1, out_dim), lambda g: (0, 0)),
        ],
        out_specs=pl.BlockSpec((CHUNK, l, out_dim), lambda g: (g, 0, 0)),
        out_shape=jax.ShapeDtypeStruct((n, l, out_dim), jnp.float32),
    )(h1f, h1b, counts, woT, bo)


def kernel(xtoken_seq, char_seq, target_chars, num_tokens, max_form_len,
           max_num_tags, eos_id, sep_id, params):
    p = params
    tc = target_chars.astype(jnp.int32)
    n, l = tc.shape
    eos_a = jnp.asarray(eos_id, jnp.int32).reshape(1)
    sep_a = jnp.asarray(sep_id, jnp.int32).reshape(1)

    ct = p['char_table']
    w0fT = p['l0_f_Wih'].T
    w0rT = p['l0_r_Wih'].T
    b0f = (p['l0_f_bih'] + p['l0_f_bhh']).reshape(1, -1)
    b0r = (p['l0_r_bih'] + p['l0_r_bhh']).reshape(1, -1)

    counts, ids, cs_tab, ptf, ptr = _run_prep(
        tc, eos_a, sep_a, ct, p['W_char'], w0fT, w0rT, b0f, b0r)

    scores_flat = _morph_scores_sc(cs_tab, tc.reshape(-1))
    morph_scores = scores_flat.reshape(n, l, -1)

    wh0cat = jnp.concatenate([p['l0_f_Whh'].T, p['l0_r_Whh'].T], axis=1)
    h0f, h0b = _run_l0(ids, counts, ptf, ptr, wh0cat)

    b1f = (p['l1_f_bih'] + p['l1_f_bhh']).reshape(1, -1)
    b1r = (p['l1_r_bih'] + p['l1_r_bhh']).reshape(1, -1)
    wh1cat = jnp.concatenate([p['l1_f_Whh'].T, p['l1_r_Whh'].T], axis=1)
    h1f, h1b = _run_l1(h0f, h0b, counts,
                       p['l1_f_Wih'].T, p['l1_r_Wih'].T, b1f, b1r, wh1cat)

    padded = _run_proj(h1f, h1b, counts, p['W_out'].T,
                       p['b_out'].reshape(1, -1))
    return morph_scores, padded
